# Initial kernel scaffold; baseline (speedup 1.0000x reference)
import jax
import jax.numpy as jnp
from jax.experimental import pallas as pl


def kernel(x, segment_idx, weight_pri, W_conv, b_conv, W_att, b_att, gn_gamma, gn_beta):
    raise NotImplementedError("write your pallas kernel here")


# speedup vs baseline: 3.7037x; 3.7037x over previous
import jax
import jax.numpy as jnp
from jax.experimental import pallas as pl
from jax.experimental.pallas import tpu as pltpu
from functools import partial

N = 320000
DF = 128
BN = 512
NB = N // BN
CV = 3 * DF
HI = jax.lax.Precision.HIGHEST


def _fwd_kernel(x_ref, idx_ref, wp_ref, wct_ref, bc_ref, wa_ref, ba_ref,
                h_ref, u_ref, p_ref, cseg_ref, cval_ref):
    i = pl.program_id(0)

    @pl.when(i == 0)
    def _():
        cseg_ref[0, 0] = -1
        cval_ref[...] = jnp.zeros((1, CV), jnp.float32)

    x = x_ref[...]
    h = jax.lax.dot_general(x, wct_ref[...], (((1,), (0,)), ((), ())),
                            precision=HI,
                            preferred_element_type=jnp.float32) + bc_ref[...]
    att = jax.lax.dot_general(h, wa_ref[...], (((1,), (0,)), ((), ())),
                              precision=HI,
                              preferred_element_type=jnp.float32) + ba_ref[0, 0]
    e = jnp.exp(att)
    u = e * wp_ref[...]
    uh = u * h
    uh2 = uh * h
    col = jax.lax.broadcasted_iota(jnp.int32, (BN, DF), 1)
    extra = jnp.where(col == 0, e, 0.0) + jnp.where(col == 1, u, 0.0)
    acc = jnp.concatenate([uh, uh2, extra], axis=1)

    sid = idx_ref[...]
    d = 1
    while d < BN:
        accs = jnp.concatenate(
            [jnp.zeros((d, CV), jnp.float32), acc[:BN - d]], axis=0)
        sids = jnp.concatenate(
            [jnp.full((d, 1), -1, jnp.int32), sid[:BN - d]], axis=0)
        acc = acc + jnp.where(sids == sid, accs, 0.0)
        d *= 2

    acc = acc + jnp.where(sid == cseg_ref[0, 0], cval_ref[...], 0.0)

    h_ref[...] = h
    u_ref[...] = u
    p_ref[...] = acc
    cseg_ref[0, 0] = idx_ref[BN - 1, 0]
    cval_ref[...] = acc[BN - 1:BN, :]


def _bwd_kernel(p_ref, idx_ref, h_ref, u_ref, mg_ref, gg_ref, gb_ref,
                out_ref, ra_ref, bseg_ref, bT_ref):
    i = pl.program_id(0)

    @pl.when(i == 0)
    def _():
        bseg_ref[0, 0] = -2
        bT_ref[...] = jnp.zeros((1, CV), jnp.float32)

    sid = idx_ref[...]
    P = p_ref[...]
    bseg = bseg_ref[0, 0]
    bT = bT_ref[...]

    next_sid = jnp.concatenate(
        [sid[1:], jnp.full((1, 1), bseg, jnp.int32)], axis=0)
    valid = (sid != next_sid).astype(jnp.int32)
    T = jnp.where(valid == 1, P, 0.0)
    d = 1
    while d < BN:
        Ts = jnp.concatenate([T[d:], jnp.zeros((d, CV), jnp.float32)], axis=0)
        vs = jnp.concatenate(
            [valid[d:], jnp.zeros((d, 1), jnp.int32)], axis=0)
        ss = jnp.concatenate(
            [sid[d:], jnp.full((d, 1), -3, jnp.int32)], axis=0)
        prop = jnp.where(ss == sid, vs, 0)
        T = jnp.where(prop * (1 - valid) == 1, Ts, T)
        valid = jnp.maximum(valid, prop)
        d *= 2
    T = jnp.where(valid == 1, T, bT)

    bseg_ref[0, 0] = idx_ref[0, 0]
    bT_ref[...] = T[0:1, :]

    h = h_ref[...]
    u = u_ref[...]
    Sh = T[:, 0:DF]
    Sh2 = T[:, DF:2 * DF]
    E = T[:, 2 * DF:2 * DF + 1]
    U = T[:, 2 * DF + 1:2 * DF + 2]
    D = jnp.maximum(U, 0.001 * E)
    a = u / D
    c = U / D
    mean = Sh / D
    var = Sh2 / D - (2.0 - c) * (mean * mean)
    std = jnp.sqrt(var + 0.001)
    outn = (h - mean) / std

    mg = mg_ref[...]
    gs = jax.lax.dot_general(outn, mg, (((1,), (0,)), ((), ())),
                             precision=HI,
                             preferred_element_type=jnp.float32) * 0.25
    gss = jax.lax.dot_general(outn * outn, mg, (((1,), (0,)), ((), ())),
                              precision=HI,
                              preferred_element_type=jnp.float32) * 0.25
    gvar = gss - gs * gs
    og = (outn - gs) * jax.lax.rsqrt(gvar + 1e-5)
    out = og * gg_ref[...] + gb_ref[...]
    out_ref[...] = jnp.maximum(out, 0.0)
    ra_ref[...] = a


@jax.jit
def kernel(x, segment_idx, weight_pri, W_conv, b_conv, W_att, b_att,
           gn_gamma, gn_beta):
    idx = segment_idx.astype(jnp.int32).reshape(N, 1)
    wp = weight_pri.reshape(N, 1)
    wct = W_conv.T
    bc = b_conv.reshape(1, DF)
    wa = W_att.reshape(DF, 1)
    ba = b_att.reshape(1, 1)
    gg = gn_gamma.reshape(1, DF)
    gb = gn_beta.reshape(1, DF)
    gidx = jnp.arange(DF) // 4
    mg = (gidx[:, None] == gidx[None, :]).astype(jnp.float32)

    row = lambda i: (i, 0)
    rep = lambda i: (0, 0)

    h, u, P = pl.pallas_call(
        _fwd_kernel,
        grid=(NB,),
        in_specs=[
            pl.BlockSpec((BN, DF), row),
            pl.BlockSpec((BN, 1), row),
            pl.BlockSpec((BN, 1), row),
            pl.BlockSpec((DF, DF), rep),
            pl.BlockSpec((1, DF), rep),
            pl.BlockSpec((DF, 1), rep),
            pl.BlockSpec((1, 1), rep),
        ],
        out_specs=[
            pl.BlockSpec((BN, DF), row),
            pl.BlockSpec((BN, 1), row),
            pl.BlockSpec((BN, CV), row),
        ],
        out_shape=[
            jax.ShapeDtypeStruct((N, DF), jnp.float32),
            jax.ShapeDtypeStruct((N, 1), jnp.float32),
            jax.ShapeDtypeStruct((N, CV), jnp.float32),
        ],
        scratch_shapes=[
            pltpu.SMEM((1, 1), jnp.int32),
            pltpu.VMEM((1, CV), jnp.float32),
        ],
    )(x, idx, wp, wct, bc, wa, ba)

    rev = lambda i: (NB - 1 - i, 0)
    out, ra = pl.pallas_call(
        _bwd_kernel,
        grid=(NB,),
        in_specs=[
            pl.BlockSpec((BN, CV), rev),
            pl.BlockSpec((BN, 1), rev),
            pl.BlockSpec((BN, DF), rev),
            pl.BlockSpec((BN, 1), rev),
            pl.BlockSpec((DF, DF), rep),
            pl.BlockSpec((1, DF), rep),
            pl.BlockSpec((1, DF), rep),
        ],
        out_specs=[
            pl.BlockSpec((BN, DF), rev),
            pl.BlockSpec((BN, 1), rev),
        ],
        out_shape=[
            jax.ShapeDtypeStruct((N, DF), jnp.float32),
            jax.ShapeDtypeStruct((N, 1), jnp.float32),
        ],
        scratch_shapes=[
            pltpu.SMEM((1, 1), jnp.int32),
            pltpu.VMEM((1, CV), jnp.float32),
        ],
    )(P, idx, h, u, mg, gg, gb)

    return out, ra
